# baseline (speedup 1.0000x reference)
import functools

import jax
import jax.numpy as jnp
from jax import lax
from jax.experimental import pallas as pl
from jax.experimental.pallas import tpu as pltpu
from jax.experimental.pallas import tpu_sc as plsc

V = 10000
E = 320000
D = 128
L = 16
NC = 2
NS = 16
NW = NC * NS
VP = 10240

CHUNK = 64
NCHUNK = E // CHUNK
IBLK = 8
NBLK = NCHUNK // IBLK
BPW_LO = NBLK // NW
NBEXTRA = NBLK - BPW_LO * NW
ROWS_PT = VP // NS

BLK = 2000


def _pre_body(v_ref, w1at_ref, w1bt_ref, b1h_ref, a_ref, b_ref):
    v = v_ref[...]
    bh = b1h_ref[...]
    a_ref[...] = jnp.dot(v, w1at_ref[...], preferred_element_type=jnp.float32) + bh
    b_ref[...] = jnp.dot(v, w1bt_ref[...], preferred_element_type=jnp.float32) + bh


def _post_body(h0_ref, h1_ref, d0_ref, d1_ref, v_ref, w2t_ref, b2_ref,
               w3at_ref, w3bt_ref, b3_ref, w4t_ref, b4_ref, o_ref):
    hs = h0_ref[0] + h1_ref[0]
    deg = d0_ref[0] + d1_ref[0]
    agg = (jnp.dot(hs, w2t_ref[...], preferred_element_type=jnp.float32)
           + deg * b2_ref[...])
    t = (jnp.dot(v_ref[...], w3at_ref[...], preferred_element_type=jnp.float32)
         + jnp.dot(agg, w3bt_ref[...], preferred_element_type=jnp.float32)
         + b3_ref[...])
    h2 = jnp.maximum(t, 0.0)
    o_ref[...] = (jnp.dot(h2, w4t_ref[...], preferred_element_type=jnp.float32)
                  + b4_ref[...])


def _edge_body(idx_hbm, a_hbm, b_hbm, h_out, deg_out,
               ib0, ib1, sc0, sc1, a0, b0, a1, b1, vbuf,
               ones_buf, degv, h_sh, deg_sh,
               gsa0, gsb0, gsa1, gsb1, ss0, ss1, ds0, ds1,
               if0, if1, zsem):
    c = lax.axis_index("c")
    s = lax.axis_index("s")
    wid = s * NC + c

    iblk = (ib0, ib1)
    idx_sc = (sc0, sc1)
    a_bufs = (a0, a1)
    b_bufs = (b0, b1)
    gsems = ((gsa0, gsb0), (gsa1, gsb1))
    ssems = (ss0, ss1)
    dsems = (ds0, ds1)
    isems = (if0, if1)

    bbase = BPW_LO * wid + jnp.minimum(wid, NBEXTRA)

    def fetch_start(bb, q, sync=False):
        if sync:
            pltpu.sync_copy(idx_hbm.at[bbase + bb], iblk[q])
        else:
            pltpu.async_copy(idx_hbm.at[bbase + bb], iblk[q], isems[q])

    def fetch_wait(bb, q):
        pltpu.make_async_copy(idx_hbm.at[bbase + bb], iblk[q],
                              isems[q]).wait()

    def gathers_start(q, j, p):
        pltpu.async_copy(a_hbm.at[iblk[q].at[j, pl.ds(0, CHUNK)]],
                         a_bufs[p], gsems[p][0])
        pltpu.async_copy(b_hbm.at[iblk[q].at[j, pl.ds(CHUNK, CHUNK)]],
                         b_bufs[p], gsems[p][1])

    def gathers_wait(p):
        pltpu.make_async_copy(a_hbm.at[iblk[0].at[0, pl.ds(0, CHUNK)]],
                              a_bufs[p], gsems[p][0]).wait()
        pltpu.make_async_copy(b_hbm.at[iblk[0].at[0, pl.ds(CHUNK, CHUNK)]],
                              b_bufs[p], gsems[p][1]).wait()

    def scatters_start(p):
        pltpu.async_copy(vbuf, h_sh.at[idx_sc[p]], ssems[p], add=True)
        pltpu.async_copy(ones_buf, deg_sh.at[idx_sc[p]], dsems[p], add=True)

    def scatters_wait(p):
        pltpu.make_async_copy(vbuf, h_sh.at[idx_sc[p]], ssems[p]).wait()
        pltpu.make_async_copy(ones_buf, deg_sh.at[idx_sc[p]], dsems[p]).wait()

    fetch_start(0, 0, sync=True)
    fetch_start(1, 1)
    gathers_start(0, 0, 0)

    zero16 = jnp.zeros((L,), jnp.float32)
    one16 = jnp.full((L,), 1.0, jnp.float32)
    for r in range(L):
        for k in range(D // L):
            vbuf[r, pl.ds(k * L, L)] = zero16
    for k in range(CHUNK // L):
        ones_buf[pl.ds(k * L, L)] = one16

    def zero_degv(r, _):
        degv[pl.ds(r * L, L)] = zero16
        return 0
    lax.fori_loop(0, ROWS_PT // L, zero_degv, 0)

    def zero_rows(r, _):
        pltpu.async_copy(vbuf.at[pl.ds(0, L)],
                         h_sh.at[pl.ds(s * ROWS_PT + r * L, L)], zsem)
        return 0
    lax.fori_loop(0, ROWS_PT // L, zero_rows, 0)
    pltpu.sync_copy(degv, deg_sh.at[pl.ds(s * ROWS_PT, ROWS_PT)])

    def zero_drain(r, _):
        pltpu.make_async_copy(vbuf.at[pl.ds(0, L)],
                              h_sh.at[pl.ds(s * ROWS_PT + r * L, L)],
                              zsem).wait()
        return 0
    lax.fori_loop(0, ROWS_PT // L, zero_drain, 0)
    plsc.subcore_barrier()

    def run_block(bb, q, fetch_next, last_block, first_block=False):
        for j in range(IBLK):
            p = j % 2
            gathers_wait(p)
            for k in range(CHUNK // L):
                idx_sc[p][pl.ds(k * L, L)] = iblk[q][j, pl.ds(CHUNK + k * L,
                                                              L)]
            if j == 0 and fetch_next:
                fetch_start(bb + 1, 1 - q)
            if j < IBLK - 1:
                gathers_start(q, j + 1, 1 - p)
            elif not last_block:
                fetch_wait(bb + 1, 1 - q)
                gathers_start(1 - q, 0, 1 - p)
            if not (first_block and j == 0):
                scatters_wait(1 - p)

            def relu_row(i, _):
                for k in range(D // L):
                    av = a_bufs[p][i, pl.ds(k * L, L)]
                    bv = b_bufs[p][i, pl.ds(k * L, L)]
                    vbuf[i, pl.ds(k * L, L)] = jnp.maximum(av + bv, 0.0)
                return 0
            lax.fori_loop(0, CHUNK, relu_row, 0)

            scatters_start(p)

    run_block(jnp.int32(0), 0, False, False, first_block=True)
    run_block(jnp.int32(1), 1, True, False)

    def block_pair(i2, _):
        run_block(2 * i2, 0, True, False)
        run_block(2 * i2 + 1, 1, True, False)
        return 0
    lax.fori_loop(1, BPW_LO // 2, block_pair, 0)

    @pl.when(wid < NBEXTRA)
    def _():
        run_block(jnp.int32(BPW_LO - 1), 0, True, False)
        run_block(jnp.int32(BPW_LO), 1, False, True)

    @pl.when(wid >= NBEXTRA)
    def _():
        run_block(jnp.int32(BPW_LO - 1), 0, False, True)
    scatters_wait(1)
    plsc.subcore_barrier()

    NDUMP = ROWS_PT // CHUNK

    def din(r, p):
        pltpu.async_copy(h_sh.at[pl.ds(s * ROWS_PT + r * CHUNK, CHUNK)],
                         a_bufs[p], gsems[p][0])

    def din_wait(r, p):
        pltpu.make_async_copy(h_sh.at[pl.ds(s * ROWS_PT + r * CHUNK, CHUNK)],
                              a_bufs[p], gsems[p][0]).wait()

    def dout(r, p):
        pltpu.async_copy(a_bufs[p], h_out.at[c, pl.ds(s * ROWS_PT + r * CHUNK,
                                                      CHUNK)], gsems[p][1])

    def dout_wait(r, p):
        pltpu.make_async_copy(a_bufs[p],
                              h_out.at[c, pl.ds(s * ROWS_PT + r * CHUNK,
                                                CHUNK)], gsems[p][1]).wait()

    din(0, 0)

    def dump_phase(r, p):
        din_wait(r, p)

        @pl.when(r + 1 < NDUMP)
        def _():
            @pl.when(r >= 1)
            def _():
                dout_wait(r - 1, 1 - p)
            din(r + 1, 1 - p)
        dout(r, p)

    def dump_wave(r2, _):
        dump_phase(2 * r2, 0)
        dump_phase(2 * r2 + 1, 1)
        return 0
    lax.fori_loop(0, NDUMP // 2, dump_wave, 0)
    dout_wait(NDUMP - 1, 1)
    pltpu.sync_copy(deg_sh.at[pl.ds(s * ROWS_PT, ROWS_PT)], degv)
    pltpu.sync_copy(degv, deg_out.at[c, pl.ds(s * ROWS_PT, ROWS_PT)])


_edge_kernel = functools.partial(
    pl.kernel,
    out_type=(
        jax.ShapeDtypeStruct((NC, VP, D), jnp.float32),
        jax.ShapeDtypeStruct((NC, VP), jnp.float32),
    ),
    mesh=plsc.VectorSubcoreMesh(core_axis_name="c", subcore_axis_name="s"),
    scratch_types=[
        pltpu.VMEM((IBLK, 2 * CHUNK), jnp.int32),
        pltpu.VMEM((IBLK, 2 * CHUNK), jnp.int32),
        pltpu.VMEM((CHUNK,), jnp.int32),
        pltpu.VMEM((CHUNK,), jnp.int32),
        pltpu.VMEM((CHUNK, D), jnp.float32),
        pltpu.VMEM((CHUNK, D), jnp.float32),
        pltpu.VMEM((CHUNK, D), jnp.float32),
        pltpu.VMEM((CHUNK, D), jnp.float32),
        pltpu.VMEM((CHUNK, D), jnp.float32),
        pltpu.VMEM((CHUNK,), jnp.float32),
        pltpu.VMEM((ROWS_PT,), jnp.float32),
        pltpu.VMEM_SHARED((VP, D), jnp.float32),
        pltpu.VMEM_SHARED((VP,), jnp.float32),
    ] + [pltpu.SemaphoreType.DMA] * 11,
)(_edge_body)


@jax.jit
def kernel(vertices, edges, W1, b1, W2, b2, W3, b3, W4, b4):
    edges32 = edges.astype(jnp.int32)
    idx_blocks = jnp.concatenate(
        [edges32[0].reshape(NCHUNK, CHUNK), edges32[1].reshape(NCHUNK, CHUNK)],
        axis=1).reshape(NBLK, IBLK, 2 * CHUNK)

    w1at = W1[:, :D].T
    w1bt = W1[:, D:].T
    b1h = (0.5 * b1)[None, :]
    w3at = W3[:, :D].T
    w3bt = W3[:, D:].T

    grid = (V // BLK,)
    row_blk = pl.BlockSpec((BLK, D), lambda i: (i, 0))
    full_w = pl.BlockSpec((D, D), lambda i: (0, 0))
    full_b = pl.BlockSpec((1, D), lambda i: (0, 0))

    a_mat, b_mat = pl.pallas_call(
        _pre_body,
        grid=grid,
        in_specs=[row_blk, full_w, full_w, full_b],
        out_specs=[row_blk, row_blk],
        out_shape=[jax.ShapeDtypeStruct((V, D), jnp.float32)] * 2,
    )(vertices, w1at, w1bt, b1h)

    h_partials, deg_partials = _edge_kernel(idx_blocks, a_mat, b_mat)
    deg3 = deg_partials[:, :, None]

    h0_blk = pl.BlockSpec((1, BLK, D), lambda i: (0, i, 0))
    h1_blk = pl.BlockSpec((1, BLK, D), lambda i: (1, i, 0))
    d0_blk = pl.BlockSpec((1, BLK, 1), lambda i: (0, i, 0))
    d1_blk = pl.BlockSpec((1, BLK, 1), lambda i: (1, i, 0))
    out = pl.pallas_call(
        _post_body,
        grid=grid,
        in_specs=[h0_blk, h1_blk, d0_blk, d1_blk, row_blk, full_w, full_b,
                  full_w, full_w, full_b, full_w, full_b],
        out_specs=row_blk,
        out_shape=jax.ShapeDtypeStruct((V, D), jnp.float32),
    )(h_partials, h_partials, deg3, deg3, vertices, W2.T, b2[None, :],
      w3at, w3bt, b3[None, :], W4.T, b4[None, :])
    return out


# speedup vs baseline: 1.0459x; 1.0459x over previous
import functools

import jax
import jax.numpy as jnp
from jax import lax
from jax.experimental import pallas as pl
from jax.experimental.pallas import tpu as pltpu
from jax.experimental.pallas import tpu_sc as plsc

V = 10000
E = 320000
D = 128
L = 16
NC = 2
NS = 16
NW = NC * NS
VP = 10240

CHUNK = 64
NCHUNK = E // CHUNK
CPW_LO = NCHUNK // NW
NEXTRA = NCHUNK - CPW_LO * NW
WAVES = CPW_LO // 2
ROWS_PT = VP // NS

BLK = 2000


def _pre_body(v_ref, w1at_ref, w1bt_ref, b1h_ref, a_ref, b_ref):
    v = v_ref[...]
    bh = b1h_ref[...]
    a_ref[...] = jnp.dot(v, w1at_ref[...], preferred_element_type=jnp.float32) + bh
    b_ref[...] = jnp.dot(v, w1bt_ref[...], preferred_element_type=jnp.float32) + bh


def _post_body(h0_ref, h1_ref, d0_ref, d1_ref, v_ref, w2t_ref, b2_ref,
               w3at_ref, w3bt_ref, b3_ref, w4t_ref, b4_ref, o_ref):
    hs = h0_ref[0] + h1_ref[0]
    deg = d0_ref[0] + d1_ref[0]
    agg = (jnp.dot(hs, w2t_ref[...], preferred_element_type=jnp.float32)
           + deg * b2_ref[...])
    t = (jnp.dot(v_ref[...], w3at_ref[...], preferred_element_type=jnp.float32)
         + jnp.dot(agg, w3bt_ref[...], preferred_element_type=jnp.float32)
         + b3_ref[...])
    h2 = jnp.maximum(t, 0.0)
    o_ref[...] = (jnp.dot(h2, w4t_ref[...], preferred_element_type=jnp.float32)
                  + b4_ref[...])


def _edge_body(src_hbm, dst_hbm, a_hbm, b_hbm, h_out, deg_out,
               is0, is1, id0, id1, sc0, sc1, a0, b0, a1, b1, vbuf,
               ones_buf, degv, h_sh, deg_sh,
               gsa0, gsb0, gsa1, gsb1, ss0, ss1, ds0, ds1,
               isem0, isem1, idsem0, idsem1, zsem):
    c = lax.axis_index("c")
    s = lax.axis_index("s")
    wid = s * NC + c

    idx_s = (is0, is1)
    idx_d = (id0, id1)
    idx_sc = (sc0, sc1)
    a_bufs = (a0, a1)
    b_bufs = (b0, b1)
    gsems = ((gsa0, gsb0), (gsa1, gsb1))
    ssems = (ss0, ss1)
    dsems = (ds0, ds1)
    isems = ((isem0, idsem0), (isem1, idsem1))

    base = CPW_LO * wid + jnp.minimum(wid, NEXTRA)
    nw_chunks = CPW_LO + jnp.where(wid < NEXTRA, 1, 0)

    def idx_start(t, p, sync=False):
        off = (base + t) * CHUNK
        if sync:
            pltpu.sync_copy(src_hbm.at[pl.ds(off, CHUNK)], idx_s[p])
            pltpu.sync_copy(dst_hbm.at[pl.ds(off, CHUNK)], idx_d[p])
        else:
            pltpu.async_copy(src_hbm.at[pl.ds(off, CHUNK)], idx_s[p],
                             isems[p][0])
            pltpu.async_copy(dst_hbm.at[pl.ds(off, CHUNK)], idx_d[p],
                             isems[p][1])

    def idx_wait(t, p):
        off = (base + t) * CHUNK
        pltpu.make_async_copy(src_hbm.at[pl.ds(off, CHUNK)], idx_s[p],
                              isems[p][0]).wait()
        pltpu.make_async_copy(dst_hbm.at[pl.ds(off, CHUNK)], idx_d[p],
                              isems[p][1]).wait()

    def gathers_start(p):
        pltpu.async_copy(a_hbm.at[idx_s[p]], a_bufs[p], gsems[p][0])
        pltpu.async_copy(b_hbm.at[idx_d[p]], b_bufs[p], gsems[p][1])

    def gathers_wait(p):
        pltpu.make_async_copy(a_hbm.at[idx_s[p]], a_bufs[p],
                              gsems[p][0]).wait()
        pltpu.make_async_copy(b_hbm.at[idx_d[p]], b_bufs[p],
                              gsems[p][1]).wait()

    def scatters_start(p):
        pltpu.async_copy(vbuf, h_sh.at[idx_sc[p]], ssems[p], add=True)
        pltpu.async_copy(ones_buf, deg_sh.at[idx_sc[p]], dsems[p], add=True)

    def scatters_wait(p):
        pltpu.make_async_copy(vbuf, h_sh.at[idx_sc[p]], ssems[p]).wait()
        pltpu.make_async_copy(ones_buf, deg_sh.at[idx_sc[p]], dsems[p]).wait()

    idx_start(0, 0, sync=True)
    gathers_start(0)
    idx_start(1, 1)

    zero16 = jnp.zeros((L,), jnp.float32)
    one16 = jnp.full((L,), 1.0, jnp.float32)
    for r in range(L):
        for k in range(D // L):
            vbuf[r, pl.ds(k * L, L)] = zero16
    for k in range(CHUNK // L):
        ones_buf[pl.ds(k * L, L)] = one16

    def zero_degv(r, _):
        degv[pl.ds(r * L, L)] = zero16
        return 0
    lax.fori_loop(0, ROWS_PT // L, zero_degv, 0)

    def zero_rows(r, _):
        pltpu.async_copy(vbuf.at[pl.ds(0, L)],
                         h_sh.at[pl.ds(s * ROWS_PT + r * L, L)], zsem)
        return 0
    lax.fori_loop(0, ROWS_PT // L, zero_rows, 0)
    pltpu.sync_copy(degv, deg_sh.at[pl.ds(s * ROWS_PT, ROWS_PT)])

    def zero_drain(r, _):
        pltpu.make_async_copy(vbuf.at[pl.ds(0, L)],
                              h_sh.at[pl.ds(s * ROWS_PT + r * L, L)],
                              zsem).wait()
        return 0
    lax.fori_loop(0, ROWS_PT // L, zero_drain, 0)
    plsc.subcore_barrier()

    def phase(t, p):
        gathers_wait(p)
        for k in range(CHUNK // L):
            idx_sc[p][pl.ds(k * L, L)] = idx_d[p][pl.ds(k * L, L)]

        @pl.when(t + 2 < nw_chunks)
        def _():
            idx_start(t + 2, p)

        @pl.when(t + 1 < nw_chunks)
        def _():
            idx_wait(t + 1, 1 - p)
            gathers_start(1 - p)

        @pl.when(t >= 1)
        def _():
            scatters_wait(1 - p)

        def relu_rows(i2, _):
            for dj in range(2):
                i = 2 * i2 + dj
                for k in range(D // L):
                    av = a_bufs[p][i, pl.ds(k * L, L)]
                    bv = b_bufs[p][i, pl.ds(k * L, L)]
                    vbuf[i, pl.ds(k * L, L)] = jnp.maximum(av + bv, 0.0)
            return 0
        lax.fori_loop(0, CHUNK // 2, relu_rows, 0)

        scatters_start(p)

    def wave(t2, _):
        phase(2 * t2, 0)
        phase(2 * t2 + 1, 1)
        return 0
    lax.fori_loop(0, WAVES, wave, 0)

    @pl.when(wid < NEXTRA)
    def _():
        phase(CPW_LO, 0)
        scatters_wait(0)

    @pl.when(wid >= NEXTRA)
    def _():
        scatters_wait(1)
    plsc.subcore_barrier()

    NDUMP = ROWS_PT // CHUNK

    def din(r, p):
        pltpu.async_copy(h_sh.at[pl.ds(s * ROWS_PT + r * CHUNK, CHUNK)],
                         a_bufs[p], gsems[p][0])

    def din_wait(r, p):
        pltpu.make_async_copy(h_sh.at[pl.ds(s * ROWS_PT + r * CHUNK, CHUNK)],
                              a_bufs[p], gsems[p][0]).wait()

    def dout(r, p):
        pltpu.async_copy(a_bufs[p], h_out.at[c, pl.ds(s * ROWS_PT + r * CHUNK,
                                                      CHUNK)], gsems[p][1])

    def dout_wait(r, p):
        pltpu.make_async_copy(a_bufs[p],
                              h_out.at[c, pl.ds(s * ROWS_PT + r * CHUNK,
                                                CHUNK)], gsems[p][1]).wait()

    din(0, 0)

    def dump_phase(r, p):
        din_wait(r, p)

        @pl.when(r + 1 < NDUMP)
        def _():
            @pl.when(r >= 1)
            def _():
                dout_wait(r - 1, 1 - p)
            din(r + 1, 1 - p)
        dout(r, p)

    def dump_wave(r2, _):
        dump_phase(2 * r2, 0)
        dump_phase(2 * r2 + 1, 1)
        return 0
    lax.fori_loop(0, NDUMP // 2, dump_wave, 0)
    dout_wait(NDUMP - 1, 1)
    pltpu.sync_copy(deg_sh.at[pl.ds(s * ROWS_PT, ROWS_PT)], degv)
    pltpu.sync_copy(degv, deg_out.at[c, pl.ds(s * ROWS_PT, ROWS_PT)])


_edge_kernel = functools.partial(
    pl.kernel,
    out_type=(
        jax.ShapeDtypeStruct((NC, VP, D), jnp.float32),
        jax.ShapeDtypeStruct((NC, VP), jnp.float32),
    ),
    mesh=plsc.VectorSubcoreMesh(core_axis_name="c", subcore_axis_name="s"),
    scratch_types=[
        pltpu.VMEM((CHUNK,), jnp.int32),
        pltpu.VMEM((CHUNK,), jnp.int32),
        pltpu.VMEM((CHUNK,), jnp.int32),
        pltpu.VMEM((CHUNK,), jnp.int32),
        pltpu.VMEM((CHUNK,), jnp.int32),
        pltpu.VMEM((CHUNK,), jnp.int32),
        pltpu.VMEM((CHUNK, D), jnp.float32),
        pltpu.VMEM((CHUNK, D), jnp.float32),
        pltpu.VMEM((CHUNK, D), jnp.float32),
        pltpu.VMEM((CHUNK, D), jnp.float32),
        pltpu.VMEM((CHUNK, D), jnp.float32),
        pltpu.VMEM((CHUNK,), jnp.float32),
        pltpu.VMEM((ROWS_PT,), jnp.float32),
        pltpu.VMEM_SHARED((VP, D), jnp.float32),
        pltpu.VMEM_SHARED((VP,), jnp.float32),
    ] + [pltpu.SemaphoreType.DMA] * 13,
)(_edge_body)


@jax.jit
def kernel(vertices, edges, W1, b1, W2, b2, W3, b3, W4, b4):
    edges32 = edges.astype(jnp.int32)
    src = edges32[0]
    dst = edges32[1]

    w1at = W1[:, :D].T
    w1bt = W1[:, D:].T
    b1h = (0.5 * b1)[None, :]
    w3at = W3[:, :D].T
    w3bt = W3[:, D:].T

    grid = (V // BLK,)
    row_blk = pl.BlockSpec((BLK, D), lambda i: (i, 0))
    full_w = pl.BlockSpec((D, D), lambda i: (0, 0))
    full_b = pl.BlockSpec((1, D), lambda i: (0, 0))

    a_mat, b_mat = pl.pallas_call(
        _pre_body,
        grid=grid,
        in_specs=[row_blk, full_w, full_w, full_b],
        out_specs=[row_blk, row_blk],
        out_shape=[jax.ShapeDtypeStruct((V, D), jnp.float32)] * 2,
    )(vertices, w1at, w1bt, b1h)

    h_partials, deg_partials = _edge_kernel(src, dst, a_mat, b_mat)
    deg3 = deg_partials[:, :, None]

    h0_blk = pl.BlockSpec((1, BLK, D), lambda i: (0, i, 0))
    h1_blk = pl.BlockSpec((1, BLK, D), lambda i: (1, i, 0))
    d0_blk = pl.BlockSpec((1, BLK, 1), lambda i: (0, i, 0))
    d1_blk = pl.BlockSpec((1, BLK, 1), lambda i: (1, i, 0))
    out = pl.pallas_call(
        _post_body,
        grid=grid,
        in_specs=[h0_blk, h1_blk, d0_blk, d1_blk, row_blk, full_w, full_b,
                  full_w, full_w, full_b, full_w, full_b],
        out_specs=row_blk,
        out_shape=jax.ShapeDtypeStruct((V, D), jnp.float32),
    )(h_partials, h_partials, deg3, deg3, vertices, W2.T, b2[None, :],
      w3at, w3bt, b3[None, :], W4.T, b4[None, :])
    return out
